# baseline (speedup 1.0000x reference)
import functools

import jax
import jax.numpy as jnp
from jax import lax
from jax.experimental import pallas as pl
from jax.experimental.pallas import tpu as pltpu
from jax.experimental.pallas import tpu_sc as plsc


_CHUNK = 128


def _make_sc_gather(V, D, NW, n_chunks):
    mesh = plsc.VectorSubcoreMesh(core_axis_name="c", subcore_axis_name="s")
    info = plsc.get_sparse_core_info()
    nc = info.num_cores

    @functools.partial(
        pl.kernel,
        mesh=mesh,
        out_type=jax.ShapeDtypeStruct((NW, n_chunks, _CHUNK, D), jnp.float32),
        scratch_types=[
            pltpu.VMEM((n_chunks, _CHUNK), jnp.int32),
            pltpu.VMEM((n_chunks, _CHUNK, D), jnp.float32),
            pltpu.SemaphoreType.DMA,
        ],
        compiler_params=pltpu.CompilerParams(use_tc_tiling_on_sc=False),
    )
    def gather_kernel(table_hbm, idx_hbm, out_hbm, idx_v, rows_v, sem):
        wid = lax.axis_index("s") * nc + lax.axis_index("c")
        pltpu.sync_copy(idx_hbm.at[wid], idx_v)
        copies = [
            pltpu.async_copy(table_hbm.at[idx_v.at[j]], rows_v.at[j], sem)
            for j in range(n_chunks)
        ]
        for c in copies:
            c.wait()
        pltpu.sync_copy(rows_v, out_hbm.at[wid])

    return gather_kernel


_VT = 512
_NBUF = 8


def _make_mm_t(B, K, V):
    nfull = V // _VT
    rem = V - nfull * _VT
    nv = nfull + (1 if rem else 0)

    def body(w_hbm, a_ref, b_ref, o_hbm, w_buf, o_buf, w_sem, o_sem):
        i = pl.program_id(0)
        slot = lax.rem(i, _NBUF)

        def w_copy(j, slot_j):
            return pltpu.make_async_copy(
                w_hbm.at[pl.ds(j * _VT, _VT), :],
                w_buf.at[slot_j],
                w_sem.at[slot_j],
            )

        def w_copy_rem(slot_j):
            return pltpu.make_async_copy(
                w_hbm.at[pl.ds(nfull * _VT, rem), :],
                w_buf.at[slot_j, pl.ds(0, rem), :],
                w_sem.at[slot_j],
            )

        def start_w(j, slot_j):
            @pl.when(j < nfull)
            def _():
                w_copy(j, slot_j).start()
            if rem:
                @pl.when(j == nfull)
                def _():
                    w_copy_rem(slot_j).start()

        def wait_w(j, slot_j):
            @pl.when(j < nfull)
            def _():
                w_copy(j, slot_j).wait()
            if rem:
                @pl.when(j == nfull)
                def _():
                    w_copy_rem(slot_j).wait()

        def o_copy(j, slot_j):
            return pltpu.make_async_copy(
                o_buf.at[slot_j],
                o_hbm.at[pl.ds(j * _VT, _VT), :],
                o_sem.at[slot_j],
            )

        def o_copy_rem(slot_j):
            return pltpu.make_async_copy(
                o_buf.at[slot_j, pl.ds(0, rem), :],
                o_hbm.at[pl.ds(nfull * _VT, rem), :],
                o_sem.at[slot_j],
            )

        def start_o(j, slot_j):
            @pl.when(j < nfull)
            def _():
                o_copy(j, slot_j).start()
            if rem:
                @pl.when(j == nfull)
                def _():
                    o_copy_rem(slot_j).start()

        def wait_o(j, slot_j):
            @pl.when(j < nfull)
            def _():
                o_copy(j, slot_j).wait()
            if rem:
                @pl.when(j == nfull)
                def _():
                    o_copy_rem(slot_j).wait()

        @pl.when(i == 0)
        def _():
            for j in range(min(_NBUF, nv)):
                start_w(j, j)

        wait_w(i, slot)

        @pl.when(i >= _NBUF)
        def _():
            wait_o(i - _NBUF, slot)

        a = jnp.maximum(a_ref[...], 0.0)
        o_buf[slot] = (
            lax.dot_general(
                w_buf[slot], a, (((1,), (1,)), ((), ())),
                preferred_element_type=jnp.float32,
            )
            + b_ref[0, pl.ds(i * _VT, _VT)][:, None]
        )

        start_o(i, slot)

        @pl.when(i + _NBUF < nv)
        def _():
            start_w(i + _NBUF, slot)

        @pl.when(i == nv - 1)
        def _():
            for k in range(min(_NBUF - 1, nv - 1)):
                j = i - (k + 1)
                wait_o(j, lax.rem(j, _NBUF))
            wait_o(i, slot)

    return pl.pallas_call(
        body,
        grid=(nv,),
        in_specs=[
            pl.BlockSpec(memory_space=pltpu.HBM),
            pl.BlockSpec((B, K), lambda i: (0, 0)),
            pl.BlockSpec((1, nv * _VT), lambda i: (0, 0)),
        ],
        out_specs=pl.BlockSpec(memory_space=pltpu.HBM),
        out_shape=jax.ShapeDtypeStruct((V, B), jnp.float32),
        scratch_shapes=[
            pltpu.VMEM((_NBUF, _VT, K), jnp.float32),
            pltpu.VMEM((_NBUF, _VT, B), jnp.float32),
            pltpu.SemaphoreType.DMA((_NBUF,)),
            pltpu.SemaphoreType.DMA((_NBUF,)),
        ],
        compiler_params=pltpu.CompilerParams(
            dimension_semantics=("arbitrary",),
            vmem_limit_bytes=100 * 1024 * 1024,
        ),
    )


def kernel(x, emb, W, b):
    B, CTX = x.shape
    V, D = emb.shape
    total = B * CTX
    NW = 32
    assert total % (NW * _CHUNK) == 0
    n_chunks = total // (NW * _CHUNK)

    idx = x.reshape(NW, n_chunks, _CHUNK).astype(jnp.int32)
    gathered = _make_sc_gather(V, D, NW, n_chunks)(emb, idx)
    a = gathered.reshape(B, CTX * D)

    nv = pl.cdiv(V, _VT)
    b_pad = jnp.zeros((1, nv * _VT), jnp.float32).at[0, :V].set(b)
    out_t = _make_mm_t(B, CTX * D, V)(W, a, b_pad)
    return out_t.T


# speedup vs baseline: 1.1030x; 1.1030x over previous
import functools

import jax
import jax.numpy as jnp
from jax import lax
from jax.experimental import pallas as pl
from jax.experimental.pallas import tpu as pltpu
from jax.experimental.pallas import tpu_sc as plsc


_CHUNK = 128


def _make_sc_gather(V, D, NW, n_chunks):
    mesh = plsc.VectorSubcoreMesh(core_axis_name="c", subcore_axis_name="s")
    info = plsc.get_sparse_core_info()
    nc = info.num_cores

    @functools.partial(
        pl.kernel,
        mesh=mesh,
        out_type=jax.ShapeDtypeStruct((NW, n_chunks, _CHUNK, D), jnp.float32),
        scratch_types=[
            pltpu.VMEM((n_chunks, _CHUNK), jnp.int32),
            pltpu.VMEM((n_chunks, _CHUNK, D), jnp.float32),
            pltpu.SemaphoreType.DMA,
        ],
        compiler_params=pltpu.CompilerParams(use_tc_tiling_on_sc=False),
    )
    def gather_kernel(table_hbm, idx_hbm, out_hbm, idx_v, rows_v, sem):
        wid = lax.axis_index("s") * nc + lax.axis_index("c")
        pltpu.sync_copy(idx_hbm.at[wid], idx_v)
        copies = [
            pltpu.async_copy(table_hbm.at[idx_v.at[j]], rows_v.at[j], sem)
            for j in range(n_chunks)
        ]
        for c in copies:
            c.wait()
        pltpu.sync_copy(rows_v, out_hbm.at[wid])

    return gather_kernel


_VT = 1536
_NBUF = 4


def _make_mm_t(B, K, V):
    nfull = V // _VT
    rem = V - nfull * _VT
    nv = nfull + (1 if rem else 0)

    def body(w_hbm, a_ref, b_ref, o_hbm, w_buf, o_buf, w_sem, o_sem):
        i = pl.program_id(0)
        slot = lax.rem(i, _NBUF)

        def w_copy(j, slot_j):
            return pltpu.make_async_copy(
                w_hbm.at[pl.ds(j * _VT, _VT), :],
                w_buf.at[slot_j],
                w_sem.at[slot_j],
            )

        def w_copy_rem(slot_j):
            return pltpu.make_async_copy(
                w_hbm.at[pl.ds(nfull * _VT, rem), :],
                w_buf.at[slot_j, pl.ds(0, rem), :],
                w_sem.at[slot_j],
            )

        def start_w(j, slot_j):
            @pl.when(j < nfull)
            def _():
                w_copy(j, slot_j).start()
            if rem:
                @pl.when(j == nfull)
                def _():
                    w_copy_rem(slot_j).start()

        def wait_w(j, slot_j):
            @pl.when(j < nfull)
            def _():
                w_copy(j, slot_j).wait()
            if rem:
                @pl.when(j == nfull)
                def _():
                    w_copy_rem(slot_j).wait()

        def o_copy(j, slot_j):
            return pltpu.make_async_copy(
                o_buf.at[slot_j],
                o_hbm.at[pl.ds(j * _VT, _VT), :],
                o_sem.at[slot_j],
            )

        def o_copy_rem(slot_j):
            return pltpu.make_async_copy(
                o_buf.at[slot_j, pl.ds(0, rem), :],
                o_hbm.at[pl.ds(nfull * _VT, rem), :],
                o_sem.at[slot_j],
            )

        def start_o(j, slot_j):
            @pl.when(j < nfull)
            def _():
                o_copy(j, slot_j).start()
            if rem:
                @pl.when(j == nfull)
                def _():
                    o_copy_rem(slot_j).start()

        def wait_o(j, slot_j):
            @pl.when(j < nfull)
            def _():
                o_copy(j, slot_j).wait()
            if rem:
                @pl.when(j == nfull)
                def _():
                    o_copy_rem(slot_j).wait()

        @pl.when(i == 0)
        def _():
            for j in range(min(_NBUF, nv)):
                start_w(j, j)

        wait_w(i, slot)

        @pl.when(i >= _NBUF)
        def _():
            wait_o(i - _NBUF, slot)

        a = jnp.maximum(a_ref[...], 0.0)
        o_buf[slot] = (
            lax.dot_general(
                w_buf[slot], a, (((1,), (1,)), ((), ())),
                preferred_element_type=jnp.float32,
            )
            + b_ref[0, pl.ds(i * _VT, _VT)][:, None]
        )

        start_o(i, slot)

        @pl.when(i + _NBUF < nv)
        def _():
            start_w(i + _NBUF, slot)

        @pl.when(i == nv - 1)
        def _():
            for k in range(min(_NBUF - 1, nv - 1)):
                j = i - (k + 1)
                wait_o(j, lax.rem(j, _NBUF))
            wait_o(i, slot)

    return pl.pallas_call(
        body,
        grid=(nv,),
        in_specs=[
            pl.BlockSpec(memory_space=pltpu.HBM),
            pl.BlockSpec((B, K), lambda i: (0, 0)),
            pl.BlockSpec((1, nv * _VT), lambda i: (0, 0)),
        ],
        out_specs=pl.BlockSpec(memory_space=pltpu.HBM),
        out_shape=jax.ShapeDtypeStruct((V, B), jnp.float32),
        scratch_shapes=[
            pltpu.VMEM((_NBUF, _VT, K), jnp.float32),
            pltpu.VMEM((_NBUF, _VT, B), jnp.float32),
            pltpu.SemaphoreType.DMA((_NBUF,)),
            pltpu.SemaphoreType.DMA((_NBUF,)),
        ],
        compiler_params=pltpu.CompilerParams(
            dimension_semantics=("arbitrary",),
            vmem_limit_bytes=100 * 1024 * 1024,
        ),
    )


def kernel(x, emb, W, b):
    B, CTX = x.shape
    V, D = emb.shape
    total = B * CTX
    NW = 32
    assert total % (NW * _CHUNK) == 0
    n_chunks = total // (NW * _CHUNK)

    idx = x.reshape(NW, n_chunks, _CHUNK).astype(jnp.int32)
    gathered = _make_sc_gather(V, D, NW, n_chunks)(emb, idx)
    a = gathered.reshape(B, CTX * D)

    nv = pl.cdiv(V, _VT)
    b_pad = jnp.zeros((1, nv * _VT), jnp.float32).at[0, :V].set(b)
    out_t = _make_mm_t(B, CTX * D, V)(W, a, b_pad)
    return out_t.T


# speedup vs baseline: 1.1054x; 1.0022x over previous
import functools

import jax
import jax.numpy as jnp
from jax import lax
from jax.experimental import pallas as pl
from jax.experimental.pallas import tpu as pltpu
from jax.experimental.pallas import tpu_sc as plsc


_CHUNK = 128


def _make_sc_gather(V, D, NW, n_chunks):
    mesh = plsc.VectorSubcoreMesh(core_axis_name="c", subcore_axis_name="s")
    info = plsc.get_sparse_core_info()
    nc = info.num_cores

    @functools.partial(
        pl.kernel,
        mesh=mesh,
        out_type=jax.ShapeDtypeStruct((NW, n_chunks, _CHUNK, D), jnp.float32),
        scratch_types=[
            pltpu.VMEM((n_chunks, _CHUNK), jnp.int32),
            pltpu.VMEM((n_chunks, _CHUNK, D), jnp.float32),
            pltpu.SemaphoreType.DMA,
        ],
        compiler_params=pltpu.CompilerParams(use_tc_tiling_on_sc=False),
    )
    def gather_kernel(table_hbm, idx_hbm, out_hbm, idx_v, rows_v, sem):
        wid = lax.axis_index("s") * nc + lax.axis_index("c")
        pltpu.sync_copy(idx_hbm.at[wid], idx_v)
        copies = [
            pltpu.async_copy(table_hbm.at[idx_v.at[j]], rows_v.at[j], sem)
            for j in range(n_chunks)
        ]
        for c in copies:
            c.wait()
        pltpu.sync_copy(rows_v, out_hbm.at[wid])

    return gather_kernel


_VT = 2048
_NBUF = 3


def _make_mm_t(B, K, V):
    nfull = V // _VT
    rem = V - nfull * _VT
    nv = nfull + (1 if rem else 0)

    def body(w_hbm, a_ref, b_ref, o_hbm, w_buf, o_buf, w_sem, o_sem):
        i = pl.program_id(0)
        slot = lax.rem(i, _NBUF)

        def w_copy(j, slot_j):
            return pltpu.make_async_copy(
                w_hbm.at[pl.ds(j * _VT, _VT), :],
                w_buf.at[slot_j],
                w_sem.at[slot_j],
            )

        def w_copy_rem(slot_j):
            return pltpu.make_async_copy(
                w_hbm.at[pl.ds(nfull * _VT, rem), :],
                w_buf.at[slot_j, pl.ds(0, rem), :],
                w_sem.at[slot_j],
            )

        def start_w(j, slot_j):
            @pl.when(j < nfull)
            def _():
                w_copy(j, slot_j).start()
            if rem:
                @pl.when(j == nfull)
                def _():
                    w_copy_rem(slot_j).start()

        def wait_w(j, slot_j):
            @pl.when(j < nfull)
            def _():
                w_copy(j, slot_j).wait()
            if rem:
                @pl.when(j == nfull)
                def _():
                    w_copy_rem(slot_j).wait()

        def o_copy(j, slot_j):
            return pltpu.make_async_copy(
                o_buf.at[slot_j],
                o_hbm.at[pl.ds(j * _VT, _VT), :],
                o_sem.at[slot_j],
            )

        def o_copy_rem(slot_j):
            return pltpu.make_async_copy(
                o_buf.at[slot_j, pl.ds(0, rem), :],
                o_hbm.at[pl.ds(nfull * _VT, rem), :],
                o_sem.at[slot_j],
            )

        def start_o(j, slot_j):
            @pl.when(j < nfull)
            def _():
                o_copy(j, slot_j).start()
            if rem:
                @pl.when(j == nfull)
                def _():
                    o_copy_rem(slot_j).start()

        def wait_o(j, slot_j):
            @pl.when(j < nfull)
            def _():
                o_copy(j, slot_j).wait()
            if rem:
                @pl.when(j == nfull)
                def _():
                    o_copy_rem(slot_j).wait()

        @pl.when(i == 0)
        def _():
            for j in range(min(_NBUF, nv)):
                start_w(j, j)

        wait_w(i, slot)

        @pl.when(i >= _NBUF)
        def _():
            wait_o(i - _NBUF, slot)

        a = jnp.maximum(a_ref[...], 0.0)
        o_buf[slot] = (
            lax.dot_general(
                w_buf[slot], a, (((1,), (1,)), ((), ())),
                preferred_element_type=jnp.float32,
            )
            + b_ref[0, pl.ds(i * _VT, _VT)][:, None]
        )

        start_o(i, slot)

        @pl.when(i + _NBUF < nv)
        def _():
            start_w(i + _NBUF, slot)

        @pl.when(i == nv - 1)
        def _():
            for k in range(min(_NBUF - 1, nv - 1)):
                j = i - (k + 1)
                wait_o(j, lax.rem(j, _NBUF))
            wait_o(i, slot)

    return pl.pallas_call(
        body,
        grid=(nv,),
        in_specs=[
            pl.BlockSpec(memory_space=pltpu.HBM),
            pl.BlockSpec((B, K), lambda i: (0, 0)),
            pl.BlockSpec((1, nv * _VT), lambda i: (0, 0)),
        ],
        out_specs=pl.BlockSpec(memory_space=pltpu.HBM),
        out_shape=jax.ShapeDtypeStruct((V, B), jnp.float32),
        scratch_shapes=[
            pltpu.VMEM((_NBUF, _VT, K), jnp.float32),
            pltpu.VMEM((_NBUF, _VT, B), jnp.float32),
            pltpu.SemaphoreType.DMA((_NBUF,)),
            pltpu.SemaphoreType.DMA((_NBUF,)),
        ],
        compiler_params=pltpu.CompilerParams(
            dimension_semantics=("arbitrary",),
            vmem_limit_bytes=100 * 1024 * 1024,
        ),
    )


def kernel(x, emb, W, b):
    B, CTX = x.shape
    V, D = emb.shape
    total = B * CTX
    NW = 32
    assert total % (NW * _CHUNK) == 0
    n_chunks = total // (NW * _CHUNK)

    idx = x.reshape(NW, n_chunks, _CHUNK).astype(jnp.int32)
    gathered = _make_sc_gather(V, D, NW, n_chunks)(emb, idx)
    a = gathered.reshape(B, CTX * D)

    nv = pl.cdiv(V, _VT)
    b_pad = jnp.zeros((1, nv * _VT), jnp.float32).at[0, :V].set(b)
    out_t = _make_mm_t(B, CTX * D, V)(W, a, b_pad)
    return out_t.T


# speedup vs baseline: 1.1137x; 1.0076x over previous
import functools

import jax
import jax.numpy as jnp
from jax import lax
from jax.experimental import pallas as pl
from jax.experimental.pallas import tpu as pltpu
from jax.experimental.pallas import tpu_sc as plsc


_CHUNK = 128


def _make_sc_gather(V, D, NW, n_chunks):
    mesh = plsc.VectorSubcoreMesh(core_axis_name="c", subcore_axis_name="s")
    info = plsc.get_sparse_core_info()
    nc = info.num_cores

    @functools.partial(
        pl.kernel,
        mesh=mesh,
        out_type=jax.ShapeDtypeStruct((NW, n_chunks, _CHUNK, D), jnp.float32),
        scratch_types=[
            pltpu.VMEM((n_chunks, _CHUNK), jnp.int32),
            pltpu.VMEM((n_chunks, _CHUNK, D), jnp.float32),
            pltpu.SemaphoreType.DMA,
        ],
        compiler_params=pltpu.CompilerParams(use_tc_tiling_on_sc=False),
    )
    def gather_kernel(table_hbm, idx_hbm, out_hbm, idx_v, rows_v, sem):
        wid = lax.axis_index("s") * nc + lax.axis_index("c")
        pltpu.sync_copy(idx_hbm.at[wid], idx_v)
        copies = [
            pltpu.async_copy(table_hbm.at[idx_v.at[j]], rows_v.at[j], sem)
            for j in range(n_chunks)
        ]
        for c in copies:
            c.wait()
        pltpu.sync_copy(rows_v, out_hbm.at[wid])

    return gather_kernel


_VT = 2048
_NBUF = 4


def _make_mm_t(B, K, V):
    nfull = V // _VT
    rem = V - nfull * _VT
    nv = nfull + (1 if rem else 0)

    def body(w_hbm, a_ref, b_ref, o_hbm, w_buf, o_buf, w_sem, o_sem):
        i = pl.program_id(0)
        slot = lax.rem(i, _NBUF)

        def w_copy(j, slot_j):
            return pltpu.make_async_copy(
                w_hbm.at[pl.ds(j * _VT, _VT), :],
                w_buf.at[slot_j],
                w_sem.at[slot_j],
            )

        def w_copy_rem(slot_j):
            return pltpu.make_async_copy(
                w_hbm.at[pl.ds(nfull * _VT, rem), :],
                w_buf.at[slot_j, pl.ds(0, rem), :],
                w_sem.at[slot_j],
            )

        def start_w(j, slot_j):
            @pl.when(j < nfull)
            def _():
                w_copy(j, slot_j).start()
            if rem:
                @pl.when(j == nfull)
                def _():
                    w_copy_rem(slot_j).start()

        def wait_w(j, slot_j):
            @pl.when(j < nfull)
            def _():
                w_copy(j, slot_j).wait()
            if rem:
                @pl.when(j == nfull)
                def _():
                    w_copy_rem(slot_j).wait()

        def o_copy(j, slot_j):
            return pltpu.make_async_copy(
                o_buf.at[slot_j],
                o_hbm.at[pl.ds(j * _VT, _VT), :],
                o_sem.at[slot_j],
            )

        def o_copy_rem(slot_j):
            return pltpu.make_async_copy(
                o_buf.at[slot_j, pl.ds(0, rem), :],
                o_hbm.at[pl.ds(nfull * _VT, rem), :],
                o_sem.at[slot_j],
            )

        def start_o(j, slot_j):
            @pl.when(j < nfull)
            def _():
                o_copy(j, slot_j).start()
            if rem:
                @pl.when(j == nfull)
                def _():
                    o_copy_rem(slot_j).start()

        def wait_o(j, slot_j):
            @pl.when(j < nfull)
            def _():
                o_copy(j, slot_j).wait()
            if rem:
                @pl.when(j == nfull)
                def _():
                    o_copy_rem(slot_j).wait()

        @pl.when(i == 0)
        def _():
            for j in range(min(_NBUF, nv)):
                start_w(j, j)

        wait_w(i, slot)

        @pl.when(i >= _NBUF)
        def _():
            wait_o(i - _NBUF, slot)

        a = jnp.maximum(a_ref[...], 0.0)
        o_buf[slot] = (
            lax.dot_general(
                w_buf[slot], a, (((1,), (1,)), ((), ())),
                preferred_element_type=jnp.float32,
            )
            + b_ref[0, pl.ds(i * _VT, _VT)][:, None]
        )

        start_o(i, slot)

        @pl.when(i + _NBUF < nv)
        def _():
            start_w(i + _NBUF, slot)

        @pl.when(i == nv - 1)
        def _():
            for k in range(min(_NBUF - 1, nv - 1)):
                j = i - (k + 1)
                wait_o(j, lax.rem(j, _NBUF))
            wait_o(i, slot)

    return pl.pallas_call(
        body,
        grid=(nv,),
        in_specs=[
            pl.BlockSpec(memory_space=pltpu.HBM),
            pl.BlockSpec((B, K), lambda i: (0, 0)),
            pl.BlockSpec((1, nv * _VT), lambda i: (0, 0)),
        ],
        out_specs=pl.BlockSpec(memory_space=pltpu.HBM),
        out_shape=jax.ShapeDtypeStruct((V, B), jnp.float32),
        scratch_shapes=[
            pltpu.VMEM((_NBUF, _VT, K), jnp.float32),
            pltpu.VMEM((_NBUF, _VT, B), jnp.float32),
            pltpu.SemaphoreType.DMA((_NBUF,)),
            pltpu.SemaphoreType.DMA((_NBUF,)),
        ],
        compiler_params=pltpu.CompilerParams(
            dimension_semantics=("arbitrary",),
            vmem_limit_bytes=100 * 1024 * 1024,
        ),
    )


def kernel(x, emb, W, b):
    B, CTX = x.shape
    V, D = emb.shape
    total = B * CTX
    NW = 32
    assert total % (NW * _CHUNK) == 0
    n_chunks = total // (NW * _CHUNK)

    idx = x.reshape(NW, n_chunks, _CHUNK).astype(jnp.int32)
    gathered = _make_sc_gather(V, D, NW, n_chunks)(emb, idx)
    a = gathered.reshape(B, CTX * D)

    nv = pl.cdiv(V, _VT)
    b_pad = jnp.zeros((1, nv * _VT), jnp.float32).at[0, :V].set(b)
    out_t = _make_mm_t(B, CTX * D, V)(W, a, b_pad)
    return out_t.T


# speedup vs baseline: 1.1155x; 1.0016x over previous
import functools

import jax
import jax.numpy as jnp
from jax import lax
from jax.experimental import pallas as pl
from jax.experimental.pallas import tpu as pltpu
from jax.experimental.pallas import tpu_sc as plsc


_CHUNK = 128


def _make_sc_gather(V, D, NW, n_chunks):
    mesh = plsc.VectorSubcoreMesh(core_axis_name="c", subcore_axis_name="s")
    info = plsc.get_sparse_core_info()
    nc = info.num_cores

    @functools.partial(
        pl.kernel,
        mesh=mesh,
        out_type=jax.ShapeDtypeStruct((NW, n_chunks, _CHUNK, D), jnp.float32),
        scratch_types=[
            pltpu.VMEM((n_chunks, _CHUNK), jnp.int32),
            pltpu.VMEM((n_chunks, _CHUNK, D), jnp.float32),
            pltpu.SemaphoreType.DMA,
        ],
        compiler_params=pltpu.CompilerParams(use_tc_tiling_on_sc=False),
    )
    def gather_kernel(table_hbm, idx_hbm, out_hbm, idx_v, rows_v, sem):
        wid = lax.axis_index("s") * nc + lax.axis_index("c")
        pltpu.sync_copy(idx_hbm.at[wid], idx_v)
        copies = [
            pltpu.async_copy(table_hbm.at[idx_v.at[j]], rows_v.at[j], sem)
            for j in range(n_chunks)
        ]
        for c in copies:
            c.wait()
        pltpu.sync_copy(rows_v, out_hbm.at[wid])

    return gather_kernel


_VT = 2048
_NBUF = 4


def _make_mm_t(B, K, V):
    nfull = V // _VT
    rem = V - nfull * _VT
    nv = nfull + (1 if rem else 0)

    def body(w_hbm, a_ref, b_ref, o_hbm, w_buf, o_buf, w_sem, o_sem):
        i = pl.program_id(0)
        slot = lax.rem(i, _NBUF)

        def w_copy(j, slot_j):
            return pltpu.make_async_copy(
                w_hbm.at[pl.ds(j * _VT, _VT), :],
                w_buf.at[slot_j],
                w_sem.at[slot_j],
            )

        def w_copy_rem(slot_j):
            return pltpu.make_async_copy(
                w_hbm.at[pl.ds(nfull * _VT, rem), :],
                w_buf.at[slot_j, pl.ds(0, rem), :],
                w_sem.at[slot_j],
            )

        def start_w(j, slot_j):
            @pl.when(j < nfull)
            def _():
                w_copy(j, slot_j).start()
            if rem:
                @pl.when(j == nfull)
                def _():
                    w_copy_rem(slot_j).start()

        def wait_w(j, slot_j):
            @pl.when(j < nfull)
            def _():
                w_copy(j, slot_j).wait()
            if rem:
                @pl.when(j == nfull)
                def _():
                    w_copy_rem(slot_j).wait()

        def o_copy(j, slot_j, h):
            hh = _VT // 2
            return pltpu.make_async_copy(
                o_buf.at[slot_j, pl.ds(h * hh, hh), :],
                o_hbm.at[pl.ds(j * _VT + h * hh, hh), :],
                o_sem.at[slot_j, h],
            )

        def o_copy_rem(slot_j):
            return pltpu.make_async_copy(
                o_buf.at[slot_j, pl.ds(0, rem), :],
                o_hbm.at[pl.ds(nfull * _VT, rem), :],
                o_sem.at[slot_j, 0],
            )

        def start_o(j, slot_j):
            @pl.when(j < nfull)
            def _():
                o_copy(j, slot_j, 0).start()
                o_copy(j, slot_j, 1).start()
            if rem:
                @pl.when(j == nfull)
                def _():
                    o_copy_rem(slot_j).start()

        def wait_o(j, slot_j):
            @pl.when(j < nfull)
            def _():
                o_copy(j, slot_j, 0).wait()
                o_copy(j, slot_j, 1).wait()
            if rem:
                @pl.when(j == nfull)
                def _():
                    o_copy_rem(slot_j).wait()

        @pl.when(i == 0)
        def _():
            for j in range(min(_NBUF, nv)):
                start_w(j, j)

        wait_w(i, slot)

        @pl.when(i >= _NBUF)
        def _():
            wait_o(i - _NBUF, slot)

        a = jnp.maximum(a_ref[...], 0.0)
        o_buf[slot] = (
            lax.dot_general(
                w_buf[slot], a, (((1,), (1,)), ((), ())),
                preferred_element_type=jnp.float32,
            )
            + b_ref[0, pl.ds(i * _VT, _VT)][:, None]
        )

        start_o(i, slot)

        @pl.when(i + _NBUF < nv)
        def _():
            start_w(i + _NBUF, slot)

        @pl.when(i == nv - 1)
        def _():
            for k in range(min(_NBUF - 1, nv - 1)):
                j = i - (k + 1)
                wait_o(j, lax.rem(j, _NBUF))
            wait_o(i, slot)

    return pl.pallas_call(
        body,
        grid=(nv,),
        in_specs=[
            pl.BlockSpec(memory_space=pltpu.HBM),
            pl.BlockSpec((B, K), lambda i: (0, 0)),
            pl.BlockSpec((1, nv * _VT), lambda i: (0, 0)),
        ],
        out_specs=pl.BlockSpec(memory_space=pltpu.HBM),
        out_shape=jax.ShapeDtypeStruct((V, B), jnp.float32),
        scratch_shapes=[
            pltpu.VMEM((_NBUF, _VT, K), jnp.float32),
            pltpu.VMEM((_NBUF, _VT, B), jnp.float32),
            pltpu.SemaphoreType.DMA((_NBUF,)),
            pltpu.SemaphoreType.DMA((_NBUF, 2)),
        ],
        compiler_params=pltpu.CompilerParams(
            dimension_semantics=("arbitrary",),
            vmem_limit_bytes=100 * 1024 * 1024,
        ),
    )


def kernel(x, emb, W, b):
    B, CTX = x.shape
    V, D = emb.shape
    total = B * CTX
    NW = 32
    assert total % (NW * _CHUNK) == 0
    n_chunks = total // (NW * _CHUNK)

    idx = x.reshape(NW, n_chunks, _CHUNK).astype(jnp.int32)
    gathered = _make_sc_gather(V, D, NW, n_chunks)(emb, idx)
    a = gathered.reshape(B, CTX * D)

    nv = pl.cdiv(V, _VT)
    b_pad = jnp.zeros((1, nv * _VT), jnp.float32).at[0, :V].set(b)
    out_t = _make_mm_t(B, CTX * D, V)(W, a, b_pad)
    return out_t.T
